# Initial kernel scaffold; baseline (speedup 1.0000x reference)
import jax
import jax.numpy as jnp
from jax.experimental import pallas as pl


def kernel(x, edge_index, mask_token, in_W, in_b, ln_g, ln_b, sage_Wl, sage_bl, sage_Wr, bn_g, bn_b, gate_W, gate_b, out_W, out_b):
    raise NotImplementedError("write your pallas kernel here")


# speedup vs baseline: 7.5795x; 7.5795x over previous
import functools

import jax
import jax.numpy as jnp
from jax import lax
from jax.experimental import pallas as pl
from jax.experimental.pallas import tpu as pltpu
from jax.experimental.pallas import tpu_sc as plsc

NSC = 2
NT = 16
CH = 120


def _gelu(x):
    return 0.5 * x * (1.0 + lax.erf(x * 0.7071067811865475))


def _tc_input_body(x_ref, mt_ref, w_ref, b_ref, g_ref, bb_ref, out_ref):
    x = x_ref[...]
    n = x.shape[0]
    row0 = lax.broadcasted_iota(jnp.int32, x.shape, 0) == 0
    xm = jnp.where(row0, mt_ref[...], x)
    h = lax.dot_general(xm, w_ref[...], (((1,), (1,)), ((), ())))
    h = h + b_ref[...]
    mu = jnp.mean(h, axis=-1, keepdims=True)
    d = h - mu
    var = jnp.mean(d * d, axis=-1, keepdims=True)
    h = d * lax.rsqrt(var + 1e-5) * g_ref[...] + bb_ref[...]
    h = _gelu(h)
    hw = h.shape[1] // 2
    out_ref[0:n] = h[:, :hw]
    out_ref[n:2 * n] = h[:, hw:]


def _tc_layer_body(n, n_pad, h2_ref, agg2_ref, deg_ref, wl_ref, bl_ref,
                   wr_ref, bng_ref, bnb_ref, gwh_ref, gwc_ref, gb_ref,
                   out_ref):
    h = jnp.concatenate([h2_ref[0:n], h2_ref[n:2 * n]], axis=1)
    agg = jnp.concatenate([agg2_ref[0:n], agg2_ref[n_pad:n_pad + n]], axis=1)
    deg = jnp.maximum(deg_ref[0:n, 0:1] + deg_ref[n_pad:n_pad + n, 0:1], 1.0)
    agg = agg / deg
    c = (lax.dot_general(agg, wl_ref[...], (((1,), (1,)), ((), ())))
         + bl_ref[...]
         + lax.dot_general(h, wr_ref[...], (((1,), (1,)), ((), ()))))
    m = jnp.mean(c, axis=0, keepdims=True)
    d = c - m
    v = jnp.mean(d * d, axis=0, keepdims=True)
    cbn = d * lax.rsqrt(v + 1e-5) * bng_ref[...] + bnb_ref[...]
    cg = _gelu(cbn)
    z = (lax.dot_general(h, gwh_ref[...], (((1,), (1,)), ((), ())))
         + lax.dot_general(cg, gwc_ref[...], (((1,), (1,)), ((), ())))
         + gb_ref[...])
    gate = jax.nn.sigmoid(z)
    hn = gate * h + (1.0 - gate) * cg
    hw = hn.shape[1] // 2
    out_ref[0:n] = hn[:, :hw]
    out_ref[n:2 * n] = hn[:, hw:]


def _make_tc_out_body(num_layers, hdim):
    def body(*refs):
        h2_refs = refs[:num_layers]
        w_ref, b_ref, out_ref = refs[num_layers:]
        n = out_ref.shape[0]
        w = w_ref[...]
        acc = jnp.broadcast_to(b_ref[...], (n, w.shape[0]))
        for i, r in enumerate(h2_refs):
            hl = jnp.concatenate([r[0:n], r[n:2 * n]], axis=1)
            wsl = w[:, i * hdim:(i + 1) * hdim]
            acc = acc + lax.dot_general(hl, wsl, (((1,), (1,)), ((), ())))
        out_ref[...] = acc
    return body


def _sc_agg_body(n_nodes, n_pad, k_chunks, *refs):
    (h_hbm, src_hbm, dst_hbm, z128_hbm,
     agg_out,
     agg_sh, sidx, didx, rows,
     gsem0, gsem1, gsem2, isem0, isem1, isem2,
     ssem0, ssem1, ssem2) = refs
    c = lax.axis_index("c")
    s = lax.axis_index("s")
    zr = n_pad // NT
    gsems = (gsem0, gsem1, gsem2)
    isems = (isem0, isem1, isem2)
    ssems = (ssem0, ssem1, ssem2)
    pltpu.sync_copy(z128_hbm.at[pl.ds(s * zr, zr)], agg_sh.at[pl.ds(s * zr, zr)])
    plsc.subcore_barrier()

    nb = 3
    ns = 2 * nb

    def idx_start(j, sl):
        pltpu.async_copy(src_hbm.at[c, s, j], sidx.at[sl], isems[sl % nb])
        pltpu.async_copy(dst_hbm.at[s, j], didx.at[sl], isems[sl % nb])

    def idx_wait(j, sl):
        pltpu.make_async_copy(src_hbm.at[c, s, j], sidx.at[sl],
                              isems[sl % nb]).wait()
        pltpu.make_async_copy(dst_hbm.at[s, j], didx.at[sl],
                              isems[sl % nb]).wait()

    def gather_start(b, sl):
        pltpu.async_copy(h_hbm.at[sidx.at[sl]], rows.at[b], gsems[b])

    def gather_wait(b, sl):
        pltpu.make_async_copy(h_hbm.at[sidx.at[sl]], rows.at[b],
                              gsems[b]).wait()

    def scatter_start(b, sl):
        pltpu.async_copy(rows.at[b], agg_sh.at[didx.at[sl]], ssems[b],
                         add=True)

    def scatter_wait(b, sl):
        pltpu.make_async_copy(rows.at[b], agg_sh.at[didx.at[sl]],
                              ssems[b]).wait()

    for m in range(nb):
        idx_start(m, m)
    for m in range(nb - 1):
        idx_wait(m, m)
        gather_start(m, m)

    loop_hi = -(-k_chunks // ns) * ns

    @pl.loop(0, loop_hi, step=ns)
    def _loop(j0):
        for bb in range(ns):
            j = j0 + bb
            rb = bb % nb

            @pl.when(j + nb - 1 < k_chunks)
            def _next_gather():
                idx_wait(j + nb - 1, (bb + nb - 1) % ns)

                @pl.when(j >= 1)
                def _drain_prev_scatter():
                    scatter_wait((bb + nb - 1) % nb, (bb + ns - 1) % ns)

                gather_start((bb + nb - 1) % nb, (bb + nb - 1) % ns)

            @pl.when(j < k_chunks)
            def _body():
                gather_wait(rb, bb)
                scatter_start(rb, bb)

            @pl.when(j + nb < k_chunks)
            def _next_idx():
                idx_start(j + nb, (bb + nb) % ns)

    for m in range(k_chunks - nb, k_chunks):
        scatter_wait(m % nb, m % ns)

    plsc.subcore_barrier()
    pltpu.sync_copy(agg_sh.at[pl.ds(s * zr, zr)],
                    agg_out.at[pl.ds(c * n_pad + s * zr, zr)])


def _sc_deg_body(n_pad, k_half, *refs):
    (dst_hbm, z128_hbm, ones_hbm,
     deg_out,
     deg_sh, didx, onesv, isem0, isem1) = refs
    c = lax.axis_index("c")
    s = lax.axis_index("s")
    zr = n_pad // NT
    isems = (isem0, isem1)
    pltpu.sync_copy(z128_hbm.at[pl.ds(s * zr, zr)], deg_sh.at[pl.ds(s * zr, zr)])
    pltpu.sync_copy(ones_hbm, onesv)
    plsc.subcore_barrier()

    def didx_start(j, b):
        pltpu.async_copy(dst_hbm.at[s, c * k_half + j], didx.at[b], isems[b])

    def didx_wait(j, b):
        pltpu.make_async_copy(dst_hbm.at[s, c * k_half + j], didx.at[b],
                              isems[b]).wait()

    didx_start(0, 0)

    @pl.loop(0, k_half, step=2)
    def _deg_loop(j0):
        for bb in range(2):
            j = j0 + bb
            didx_wait(j, bb)

            @pl.when(j + 1 < k_half)
            def _dnext():
                didx_start(j + 1, 1 - bb)

            pltpu.sync_copy(onesv, deg_sh.at[didx.at[bb]], add=True)

    plsc.subcore_barrier()
    pltpu.sync_copy(deg_sh.at[pl.ds(s * zr, zr)],
                    deg_out.at[pl.ds(c * n_pad + s * zr, zr)])


@functools.lru_cache(maxsize=None)
def _make_sc_agg(n_nodes, n_pad, k_chunks, hw):
    mesh = plsc.VectorSubcoreMesh(core_axis_name="c", subcore_axis_name="s")
    out_type = jax.ShapeDtypeStruct((NSC * n_pad, hw), jnp.float32)
    scratch = [
        pltpu.VMEM_SHARED((n_pad, hw), jnp.float32),
        pltpu.VMEM((6, CH), jnp.int32),
        pltpu.VMEM((6, CH), jnp.int32),
        pltpu.VMEM((3, CH, hw), jnp.float32),
    ] + [pltpu.SemaphoreType.DMA] * 9
    return pl.kernel(
        functools.partial(_sc_agg_body, n_nodes, n_pad, k_chunks),
        out_type=out_type,
        mesh=mesh,
        scratch_types=scratch,
    )


@functools.lru_cache(maxsize=None)
def _make_sc_deg(n_pad, k_half):
    mesh = plsc.VectorSubcoreMesh(core_axis_name="c", subcore_axis_name="s")
    out_type = jax.ShapeDtypeStruct((NSC * n_pad, 128), jnp.float32)
    scratch = [
        pltpu.VMEM_SHARED((n_pad, 128), jnp.float32),
        pltpu.VMEM((2, CH), jnp.int32),
        pltpu.VMEM((CH, 128), jnp.float32),
        pltpu.SemaphoreType.DMA,
        pltpu.SemaphoreType.DMA,
    ]
    return pl.kernel(
        functools.partial(_sc_deg_body, n_pad, k_half),
        out_type=out_type,
        mesh=mesh,
        scratch_types=scratch,
    )


def _sc_aggregate(h2, src2, dst3, z128):
    n_nodes = h2.shape[0] // NSC
    hw = h2.shape[1]
    n_pad = z128.shape[0]
    k_chunks = dst3.shape[1]
    return _make_sc_agg(n_nodes, n_pad, k_chunks, hw)(h2, src2, dst3, z128)


def _sc_degree(dst3, z128, ones128):
    n_pad = z128.shape[0]
    k_half = dst3.shape[1] // NSC
    return _make_sc_deg(n_pad, k_half)(dst3, z128, ones128)


def kernel(x, edge_index, mask_token, in_W, in_b, ln_g, ln_b,
           sage_Wl, sage_bl, sage_Wr, bn_g, bn_b, gate_W, gate_b,
           out_W, out_b):
    n, in_dim = x.shape
    e = edge_index.shape[1]
    h = in_W.shape[0]
    num_layers = sage_Wl.shape[0]
    hw = h // 2
    assert n % NT == 0

    k_chunks = -(-(-(-e // (NT * CH))) // 4) * 4
    e_pad = k_chunks * CH * NT
    pad = e_pad - e
    n_pad = -(-(n + 64) // (NT * 8)) * (NT * 8)

    src = edge_index[0]
    dst = edge_index[1]
    pad_ar = jnp.arange(pad, dtype=jnp.int32)
    src_p = jnp.concatenate([src, pad_ar % n])
    dst_p = jnp.concatenate([dst, n + pad_ar % (n_pad - n)])
    src3 = src_p.reshape(NT, k_chunks, CH)
    src2 = jnp.stack([src3, src3 + n])
    dst3 = dst_p.reshape(NT, k_chunks, CH)

    z128 = jnp.zeros((n_pad, hw), jnp.float32)
    ones128 = jnp.ones((CH, 128), jnp.float32)

    tc_params = pltpu.CompilerParams(vmem_limit_bytes=100 * 1024 * 1024)
    no = jax.ShapeDtypeStruct((n, out_W.shape[0]), jnp.float32)
    tc_input = pl.pallas_call(
        _tc_input_body,
        out_shape=jax.ShapeDtypeStruct((NSC * n, hw), jnp.float32),
        compiler_params=tc_params,
    )
    tc_layer = pl.pallas_call(
        functools.partial(_tc_layer_body, n, n_pad),
        out_shape=jax.ShapeDtypeStruct((NSC * n, hw), jnp.float32),
        compiler_params=tc_params,
    )
    tc_out = pl.pallas_call(
        _make_tc_out_body(num_layers, h),
        out_shape=no,
        compiler_params=tc_params,
    )

    deg2 = _sc_degree(dst3, z128, ones128)
    h2 = tc_input(x, mask_token[None], in_W, in_b[None], ln_g[None], ln_b[None])

    outs2 = []
    for l in range(num_layers):
        agg2 = _sc_aggregate(h2, src2, dst3, z128)
        h2 = tc_layer(h2, agg2, deg2, sage_Wl[l], sage_bl[l][None],
                      sage_Wr[l], bn_g[l][None], bn_b[l][None],
                      gate_W[l, :, :h], gate_W[l, :, h:], gate_b[l][None])
        outs2.append(h2)

    return tc_out(*outs2, out_W, out_b[None])
